# baseline (speedup 1.0000x reference)
import functools

import jax
import jax.numpy as jnp
from jax import lax
from jax.experimental import pallas as pl
from jax.experimental.pallas import tpu as pltpu
from jax.experimental.pallas import tpu_sc as plsc

C = 21
NPIX = 4 * 384 * 384
PPI = 384 * 384
NC, NS, L = 2, 16, 16
NW = NC * NS
PPT = NPIX // NW
CH = 1024
NCHUNK = PPT // CH
K = 1024
ROW = 2 * K + 16
HIST = C * ROW
GROUPS = CH // L


def _sc_body(logits_hbm, labels_hbm, out_hbm, ebuf, labbuf, hist):
    wid = lax.axis_index("s") * NC + lax.axis_index("c")
    img = wid // (NW // 4)
    base = (wid % (NW // 4)) * PPT

    zeros = jnp.zeros((L,), jnp.float32)
    ones = jnp.ones((L,), jnp.float32)
    neg_ones = jnp.full((L,), -1.0, jnp.float32)
    iota = lax.iota(jnp.int32, L)
    kf = jnp.float32(K)
    kmax = jnp.full((L,), K - 1, jnp.int32)

    def zero_step(i, _):
        hist[pl.ds(i * L, L)] = zeros
        return 0

    lax.fori_loop(0, HIST // L, zero_step, 0)

    def chunk(t, _):
        off = base + t * CH
        pltpu.sync_copy(logits_hbm.at[img, :, pl.ds(off, CH)], ebuf)
        pltpu.sync_copy(labels_hbm.at[pl.ds(wid * PPT + t * CH, CH)], labbuf)

        @plsc.parallel_loop(0, GROUPS, unroll=2)
        def group(g):
            lab = labbuf[pl.ds(g * L, L)]
            es = []
            s = zeros
            for c in range(C):
                e = jnp.exp(ebuf[c, pl.ds(g * L, L)])
                es.append(e)
                s = s + e
            rcp = 1.0 / s
            sfg = zeros
            for c in range(C):
                p = es[c] * rcp
                bgm = lab != c
                b = jnp.minimum((p * kf).astype(jnp.int32), kmax)
                idx = b + (c * ROW)
                plsc.addupdate_scatter(hist, [idx], ones, mask=bgm)
                plsc.addupdate_scatter(hist, [idx + K], p, mask=bgm)
                sfg = sfg + jnp.where(bgm, zeros, p)
            plsc.addupdate_scatter(hist, [lab * ROW + (2 * K) + iota], sfg)

        return 0

    lax.fori_loop(0, NCHUNK, chunk, 0)
    pltpu.sync_copy(hist, out_hbm.at[wid])


@functools.partial(jax.jit, static_argnums=())
def _sc_hist(logits3, labels_flat):
    mesh = plsc.VectorSubcoreMesh(core_axis_name="c", subcore_axis_name="s")
    return pl.kernel(
        _sc_body,
        out_type=jax.ShapeDtypeStruct((NW, HIST), jnp.float32),
        mesh=mesh,
        compiler_params=pltpu.CompilerParams(needs_layout_passes=False),
        scratch_types=[
            pltpu.VMEM((C, CH), jnp.float32),
            pltpu.VMEM((CH,), jnp.int32),
            pltpu.VMEM((HIST,), jnp.float32),
        ],
    )(logits3, labels_flat)


def _tc_body(tab_ref, out_ref):
    t = jnp.sum(tab_ref[...], axis=0)
    counts = t[:, 0:K]
    sums = t[:, K:2 * K]
    s_fg = jnp.sum(t[:, 2 * K:], axis=1, keepdims=True)
    bg = jnp.sum(counts, axis=1, keepdims=True)
    fg = jnp.float32(NPIX) - bg
    ii = lax.broadcasted_iota(jnp.int32, (K, K), 0)
    jj = lax.broadcasted_iota(jnp.int32, (K, K), 1)
    m = (ii > jj).astype(jnp.float32)
    r = jax.lax.dot_general(counts, m, (((1,), (0,)), ((), ())),
                            preferred_element_type=jnp.float32)
    d1 = jnp.maximum(fg + r, 1.0)
    d2 = jnp.maximum(fg + r + counts, 1.0)
    t_c = fg * jnp.sum(sums / (d1 * d2), axis=1, keepdims=True)
    loss = 1.0 - s_fg / jnp.float32(NPIX) + t_c
    present = jnp.logical_and(fg > 0.5, bg > 0.5).astype(jnp.float32)
    cnt = jnp.sum(present)
    total = jnp.sum(loss * present)
    res = jnp.where(cnt > 0.0, total / cnt, 0.0)
    out_ref[...] = jnp.reshape(res, (1, 1))


def _tc_combine(table):
    return pl.pallas_call(
        _tc_body,
        out_shape=jax.ShapeDtypeStruct((1, 1), jnp.float32),
    )(table)


def kernel(logits, labels):
    logits3 = logits.reshape(4, C, PPI)
    labels_flat = labels.reshape(-1)
    table = _sc_hist(logits3, labels_flat)
    out = _tc_combine(table.reshape(NW, C, ROW))
    return out[0, 0]


# speedup vs baseline: 1.5739x; 1.5739x over previous
import functools

import jax
import jax.numpy as jnp
from jax import lax
from jax.experimental import pallas as pl
from jax.experimental.pallas import tpu as pltpu
from jax.experimental.pallas import tpu_sc as plsc

C = 21
NPIX = 4 * 384 * 384
PPI = 384 * 384
NC, NS, L = 2, 16, 16
NW = NC * NS
PPT = NPIX // NW
CH = 1024
NCHUNK = PPT // CH
K = 1024
ROW = 2 * K + 16
HIST = C * ROW
GROUPS = CH // L


def _sc_body(logits_hbm, labels_hbm, out_hbm, ebuf, labbuf, hist, sem0, sem1):
    wid = lax.axis_index("s") * NC + lax.axis_index("c")
    img = wid // (NW // 4)
    base = (wid % (NW // 4)) * PPT

    zeros = jnp.zeros((L,), jnp.float32)
    ones = jnp.ones((L,), jnp.float32)
    iota = lax.iota(jnp.int32, L)
    kf = jnp.float32(K)
    kmaxf = jnp.full((L,), K - 1, jnp.float32)
    sems = (sem0, sem1)

    def zero_step(i, _):
        for j in range(3):
            hist[pl.ds(i * (3 * L) + j * L, L)] = zeros
        return 0

    lax.fori_loop(0, HIST // (3 * L), zero_step, 0)

    def start_copies(t):
        par = t % 2
        off = base + t * CH
        ca = pltpu.make_async_copy(
            logits_hbm.at[img, :, pl.ds(off, CH)], ebuf.at[par], sems[par])
        cb = pltpu.make_async_copy(
            labels_hbm.at[pl.ds(wid * PPT + t * CH, CH)], labbuf.at[par],
            sems[par])
        ca.start()
        cb.start()
        return ca, cb

    pend = start_copies(0)
    for t in range(NCHUNK):
        par = t % 2
        pend[0].wait()
        pend[1].wait()
        if t + 1 < NCHUNK:
            pend = start_copies(t + 1)

        def group(g, _, par=par):
            lab = labbuf[par, pl.ds(g * L, L)]
            es = []
            s = zeros
            for c in range(C):
                e = jnp.exp(ebuf[par, c, pl.ds(g * L, L)])
                es.append(e)
                s = s + e
            rcp = 1.0 / s
            sfg = zeros
            for c in range(C):
                p = es[c] * rcp
                bgm = lab != c
                b = jnp.minimum(p * kf, kmaxf).astype(jnp.int32)
                idx = b + (c * ROW)
                plsc.addupdate_scatter(hist, [idx], ones, mask=bgm)
                plsc.addupdate_scatter(hist, [idx + K], p, mask=bgm)
                sfg = sfg + jnp.where(bgm, zeros, p)
            plsc.addupdate_scatter(hist, [lab * ROW + (2 * K) + iota], sfg)
            return 0

        lax.fori_loop(0, GROUPS, group, 0)

    pltpu.sync_copy(hist, out_hbm.at[wid])


@functools.partial(jax.jit, static_argnums=())
def _sc_hist(logits3, labels_flat):
    mesh = plsc.VectorSubcoreMesh(core_axis_name="c", subcore_axis_name="s")
    return pl.kernel(
        _sc_body,
        out_type=jax.ShapeDtypeStruct((NW, HIST), jnp.float32),
        mesh=mesh,
        compiler_params=pltpu.CompilerParams(needs_layout_passes=False),
        scratch_types=[
            pltpu.VMEM((2, C, CH), jnp.float32),
            pltpu.VMEM((2, CH), jnp.int32),
            pltpu.VMEM((HIST,), jnp.float32),
            pltpu.SemaphoreType.DMA,
            pltpu.SemaphoreType.DMA,
        ],
    )(logits3, labels_flat)


def _tc_body(tab_ref, out_ref):
    t = jnp.sum(tab_ref[...], axis=0)
    counts = t[:, 0:K]
    sums = t[:, K:2 * K]
    s_fg = jnp.sum(t[:, 2 * K:], axis=1, keepdims=True)
    bg = jnp.sum(counts, axis=1, keepdims=True)
    fg = jnp.float32(NPIX) - bg
    ii = lax.broadcasted_iota(jnp.int32, (K, K), 0)
    jj = lax.broadcasted_iota(jnp.int32, (K, K), 1)
    m = (ii > jj).astype(jnp.float32)
    r = jax.lax.dot_general(counts, m, (((1,), (0,)), ((), ())),
                            preferred_element_type=jnp.float32)
    d1 = jnp.maximum(fg + r, 1.0)
    d2 = jnp.maximum(fg + r + counts, 1.0)
    t_c = fg * jnp.sum(sums / (d1 * d2), axis=1, keepdims=True)
    loss = 1.0 - s_fg / jnp.float32(NPIX) + t_c
    present = jnp.logical_and(fg > 0.5, bg > 0.5).astype(jnp.float32)
    cnt = jnp.sum(present)
    total = jnp.sum(loss * present)
    res = jnp.where(cnt > 0.0, total / cnt, 0.0)
    out_ref[...] = jnp.reshape(res, (1, 1))


def _tc_combine(table):
    return pl.pallas_call(
        _tc_body,
        out_shape=jax.ShapeDtypeStruct((1, 1), jnp.float32),
    )(table)


def kernel(logits, labels):
    logits3 = logits.reshape(4, C, PPI)
    labels_flat = labels.reshape(-1)
    table = _sc_hist(logits3, labels_flat)
    out = _tc_combine(table.reshape(NW, C, ROW))
    return out[0, 0]


# speedup vs baseline: 1.5879x; 1.0089x over previous
import functools

import jax
import jax.numpy as jnp
from jax import lax
from jax.experimental import pallas as pl
from jax.experimental.pallas import tpu as pltpu
from jax.experimental.pallas import tpu_sc as plsc

C = 21
NPIX = 4 * 384 * 384
PPI = 384 * 384
NC, NS, L = 2, 16, 16
NW = NC * NS
PPT = NPIX // NW
CH = 1024
NCHUNK = PPT // CH
K = 1024
GROUPS = CH // L
CNT_W = C * K
SFG_W = C * L


def _sc_body(logits_hbm, labels_hbm, cnt_hbm, sum_hbm, sfg_hbm,
             ebuf, labbuf, cnt, summ, sfgt, sem0, sem1):
    wid = lax.axis_index("s") * NC + lax.axis_index("c")
    img = wid // (NW // 4)
    base = (wid % (NW // 4)) * PPT

    zeros = jnp.zeros((L,), jnp.float32)
    ones = jnp.ones((L,), jnp.float32)
    neg_ones = jnp.full((L,), -1.0, jnp.float32)
    iota = lax.iota(jnp.int32, L)
    kf = jnp.float32(K)
    kmaxf = jnp.full((L,), K - 1, jnp.float32)
    sems = (sem0, sem1)

    def zero_step(i, _):
        for j in range(3):
            cnt[pl.ds(i * (3 * L) + j * L, L)] = zeros
            summ[pl.ds(i * (3 * L) + j * L, L)] = zeros
        return 0

    lax.fori_loop(0, CNT_W // (3 * L), zero_step, 0)
    for j in range(SFG_W // L):
        sfgt[pl.ds(j * L, L)] = zeros

    def start_copies(t):
        par = t % 2
        off = base + t * CH
        ca = pltpu.make_async_copy(
            logits_hbm.at[img, :, pl.ds(off, CH)], ebuf.at[par], sems[par])
        cb = pltpu.make_async_copy(
            labels_hbm.at[pl.ds(wid * PPT + t * CH, CH)], labbuf.at[par],
            sems[par])
        ca.start()
        cb.start()
        return ca, cb

    pend = start_copies(0)
    for t in range(NCHUNK):
        par = t % 2
        pend[0].wait()
        pend[1].wait()
        if t + 1 < NCHUNK:
            pend = start_copies(t + 1)

        parv = jnp.full((L,), par, jnp.int32)

        def group(g, _, par=par, parv=parv):
            lab = labbuf[par, pl.ds(g * L, L)]
            es = []
            s = zeros
            for c in range(C):
                e = jnp.exp(ebuf[par, c, pl.ds(g * L, L)])
                es.append(e)
                s = s + e
            rcp = 1.0 / s
            pix = g * L + iota
            lfg = plsc.load_gather(ebuf, [parv, lab, pix])
            pfg = jnp.exp(lfg) * rcp
            fbin = jnp.minimum(pfg * kf, kmaxf).astype(jnp.int32)
            fidx = lab * K + fbin
            for c in range(C):
                p = es[c] * rcp
                b = jnp.minimum(p * kf, kmaxf).astype(jnp.int32)
                idx = b + (c * K)
                plsc.addupdate_scatter(cnt, [idx], ones)
                plsc.addupdate_scatter(summ, [idx], p)
            plsc.addupdate_scatter(cnt, [fidx], neg_ones)
            plsc.addupdate_scatter(summ, [fidx], -pfg)
            plsc.addupdate_scatter(sfgt, [lab * L + iota], pfg)
            return 0

        lax.fori_loop(0, GROUPS, group, 0)

    pltpu.sync_copy(cnt, cnt_hbm.at[wid])
    pltpu.sync_copy(summ, sum_hbm.at[wid])
    pltpu.sync_copy(sfgt, sfg_hbm.at[wid])


@functools.partial(jax.jit, static_argnums=())
def _sc_hist(logits3, labels_flat):
    mesh = plsc.VectorSubcoreMesh(core_axis_name="c", subcore_axis_name="s")
    return pl.kernel(
        _sc_body,
        out_type=(
            jax.ShapeDtypeStruct((NW, CNT_W), jnp.float32),
            jax.ShapeDtypeStruct((NW, CNT_W), jnp.float32),
            jax.ShapeDtypeStruct((NW, SFG_W), jnp.float32),
        ),
        mesh=mesh,
        compiler_params=pltpu.CompilerParams(needs_layout_passes=False),
        scratch_types=[
            pltpu.VMEM((2, C, CH), jnp.float32),
            pltpu.VMEM((2, CH), jnp.int32),
            pltpu.VMEM((CNT_W,), jnp.float32),
            pltpu.VMEM((CNT_W,), jnp.float32),
            pltpu.VMEM((SFG_W,), jnp.float32),
            pltpu.SemaphoreType.DMA,
            pltpu.SemaphoreType.DMA,
        ],
    )(logits3, labels_flat)


def _tc_body(cnt_ref, sum_ref, sfg_ref, out_ref):
    counts = jnp.sum(cnt_ref[...], axis=0)
    sums = jnp.sum(sum_ref[...], axis=0)
    s_fg = jnp.sum(jnp.sum(sfg_ref[...], axis=0),
                   axis=1, keepdims=True)
    bg = jnp.sum(counts, axis=1, keepdims=True)
    fg = jnp.float32(NPIX) - bg
    ii = lax.broadcasted_iota(jnp.int32, (K, K), 0)
    jj = lax.broadcasted_iota(jnp.int32, (K, K), 1)
    m = (ii > jj).astype(jnp.float32)
    r = jax.lax.dot_general(counts, m, (((1,), (0,)), ((), ())),
                            preferred_element_type=jnp.float32)
    d1 = jnp.maximum(fg + r, 1.0)
    d2 = jnp.maximum(fg + r + counts, 1.0)
    t_c = fg * jnp.sum(sums / (d1 * d2), axis=1, keepdims=True)
    loss = 1.0 - s_fg / jnp.float32(NPIX) + t_c
    present = jnp.logical_and(fg > 0.5, bg > 0.5).astype(jnp.float32)
    cntp = jnp.sum(present)
    total = jnp.sum(loss * present)
    res = jnp.where(cntp > 0.0, total / cntp, 0.0)
    out_ref[...] = jnp.reshape(res, (1, 1))


def _tc_combine(cnt, summ, sfg):
    return pl.pallas_call(
        _tc_body,
        out_shape=jax.ShapeDtypeStruct((1, 1), jnp.float32),
    )(cnt, summ, sfg)


def kernel(logits, labels):
    logits3 = logits.reshape(4, C, PPI)
    labels_flat = labels.reshape(-1)
    cnt, summ, sfg = _sc_hist(logits3, labels_flat)
    out = _tc_combine(cnt.reshape(NW, C, K), summ.reshape(NW, C, K),
                      sfg.reshape(NW, C, L))
    return out[0, 0]


# speedup vs baseline: 1.7626x; 1.1100x over previous
import functools

import jax
import jax.numpy as jnp
from jax import lax
from jax.experimental import pallas as pl
from jax.experimental.pallas import tpu as pltpu
from jax.experimental.pallas import tpu_sc as plsc

C = 21
NPIX = 4 * 384 * 384
PPI = 384 * 384
NC, NS, L = 2, 16, 16
NW = NC * NS
PPT = NPIX // NW
CH = 1024
NCHUNK = PPT // CH
K = 1024
GROUPS = CH // L
CNT_W = C * K
SFG_W = C * L


def _sc_body(logits_hbm, labels_hbm, cnt_hbm, sum_hbm, sfg_hbm,
             ebuf, labbuf, cnt, summ, sfgt, sem0, sem1):
    wid = lax.axis_index("s") * NC + lax.axis_index("c")
    img = wid // (NW // 4)
    base = (wid % (NW // 4)) * PPT

    zeros = jnp.zeros((L,), jnp.float32)
    ones = jnp.ones((L,), jnp.float32)
    neg_ones = jnp.full((L,), -1.0, jnp.float32)
    iota = lax.iota(jnp.int32, L)
    kf = jnp.float32(K)
    kmaxf = jnp.full((L,), K - 1, jnp.float32)
    sems = (sem0, sem1)

    def zero_step(i, _):
        for j in range(3):
            cnt[pl.ds(i * (3 * L) + j * L, L)] = zeros
            summ[pl.ds(i * (3 * L) + j * L, L)] = zeros
        return 0

    lax.fori_loop(0, CNT_W // (3 * L), zero_step, 0)
    for j in range(SFG_W // L):
        sfgt[pl.ds(j * L, L)] = zeros

    def start_copies(t):
        par = t % 2
        off = base + t * CH
        ca = pltpu.make_async_copy(
            logits_hbm.at[img, :, pl.ds(off, CH)],
            ebuf.at[par, :, pl.ds(0, CH)], sems[par])
        cb = pltpu.make_async_copy(
            labels_hbm.at[pl.ds(wid * PPT + t * CH, CH)],
            labbuf.at[par, pl.ds(0, CH)], sems[par])
        ca.start()
        cb.start()
        return ca, cb

    pend = start_copies(0)
    for t in range(NCHUNK):
        par = t % 2
        pend[0].wait()
        pend[1].wait()
        if t + 1 < NCHUNK:
            pend = start_copies(t + 1)

        parv = jnp.full((L,), par, jnp.int32)

        def load_group(gidx, par=par):
            lab = labbuf[par, pl.ds(gidx * L, L)]
            ls = tuple(ebuf[par, c, pl.ds(gidx * L, L)] for c in range(C))
            return lab, ls

        def group(g, carry, par=par, parv=parv):
            lab, ls = carry
            es = []
            s = zeros
            for c in range(C):
                e = jnp.exp(ls[c])
                es.append(e)
                s = s + e
            rcp = 1.0 / s
            pix = g * L + iota
            lfg = plsc.load_gather(ebuf, [parv, lab, pix])
            pfg = jnp.exp(lfg) * rcp
            fbin = jnp.minimum(pfg * kf, kmaxf).astype(jnp.int32)
            fidx = lab * K + fbin
            nxt = load_group(g + 1)
            for c in range(C):
                p = es[c] * rcp
                b = jnp.minimum(p * kf, kmaxf).astype(jnp.int32)
                idx = b + (c * K)
                plsc.addupdate_scatter(cnt, [idx], ones)
                plsc.addupdate_scatter(summ, [idx], p)
            plsc.addupdate_scatter(cnt, [fidx], neg_ones)
            plsc.addupdate_scatter(summ, [fidx], -pfg)
            plsc.addupdate_scatter(sfgt, [lab * L + iota], pfg)
            return nxt

        lax.fori_loop(0, GROUPS, group, load_group(0))

    pltpu.sync_copy(cnt, cnt_hbm.at[wid])
    pltpu.sync_copy(summ, sum_hbm.at[wid])
    pltpu.sync_copy(sfgt, sfg_hbm.at[wid])


@functools.partial(jax.jit, static_argnums=())
def _sc_hist(logits3, labels_flat):
    mesh = plsc.VectorSubcoreMesh(core_axis_name="c", subcore_axis_name="s")
    return pl.kernel(
        _sc_body,
        out_type=(
            jax.ShapeDtypeStruct((NW, CNT_W), jnp.float32),
            jax.ShapeDtypeStruct((NW, CNT_W), jnp.float32),
            jax.ShapeDtypeStruct((NW, SFG_W), jnp.float32),
        ),
        mesh=mesh,
        compiler_params=pltpu.CompilerParams(needs_layout_passes=False),
        scratch_types=[
            pltpu.VMEM((2, C, CH + L), jnp.float32),
            pltpu.VMEM((2, CH + L), jnp.int32),
            pltpu.VMEM((CNT_W,), jnp.float32),
            pltpu.VMEM((CNT_W,), jnp.float32),
            pltpu.VMEM((SFG_W,), jnp.float32),
            pltpu.SemaphoreType.DMA,
            pltpu.SemaphoreType.DMA,
        ],
    )(logits3, labels_flat)


def _tc_body(cnt_ref, sum_ref, sfg_ref, out_ref):
    counts = jnp.sum(cnt_ref[...], axis=0)
    sums = jnp.sum(sum_ref[...], axis=0)
    s_fg = jnp.sum(jnp.sum(sfg_ref[...], axis=0),
                   axis=1, keepdims=True)
    bg = jnp.sum(counts, axis=1, keepdims=True)
    fg = jnp.float32(NPIX) - bg
    ii = lax.broadcasted_iota(jnp.int32, (K, K), 0)
    jj = lax.broadcasted_iota(jnp.int32, (K, K), 1)
    m = (ii > jj).astype(jnp.float32)
    r = jax.lax.dot_general(counts, m, (((1,), (0,)), ((), ())),
                            preferred_element_type=jnp.float32)
    d1 = jnp.maximum(fg + r, 1.0)
    d2 = jnp.maximum(fg + r + counts, 1.0)
    t_c = fg * jnp.sum(sums / (d1 * d2), axis=1, keepdims=True)
    loss = 1.0 - s_fg / jnp.float32(NPIX) + t_c
    present = jnp.logical_and(fg > 0.5, bg > 0.5).astype(jnp.float32)
    cntp = jnp.sum(present)
    total = jnp.sum(loss * present)
    res = jnp.where(cntp > 0.0, total / cntp, 0.0)
    out_ref[...] = jnp.reshape(res, (1, 1))


def _tc_combine(cnt, summ, sfg):
    return pl.pallas_call(
        _tc_body,
        out_shape=jax.ShapeDtypeStruct((1, 1), jnp.float32),
    )(cnt, summ, sfg)


def kernel(logits, labels):
    logits3 = logits.reshape(4, C, PPI)
    labels_flat = labels.reshape(-1)
    cnt, summ, sfg = _sc_hist(logits3, labels_flat)
    out = _tc_combine(cnt.reshape(NW, C, K), summ.reshape(NW, C, K),
                      sfg.reshape(NW, C, L))
    return out[0, 0]
